# baseline (device time: 18873 ns/iter reference)
import jax
import jax.numpy as jnp
from jax import lax
from jax.experimental import pallas as pl
from jax.experimental.pallas import tpu as pltpu

N_DEV = 8
M_PER = 128
H = 64
K = 1024
N_PER = 128

SX, SY, SZ, SBD, SXY, SYZ, SXZ = range(7)


def _gelu(y):
    c = 0.7978845608028654
    return 0.5 * y * (1.0 + jnp.tanh(c * (y + 0.044715 * y * y * y)))


def kernel(x, w_mat):
    def body(x_ref, w_ref, out_ref, own_ref, recv_ref, send_sems, recv_sems):
        my = lax.axis_index("i")

        z = my // 4
        p = my % 4
        y = p // 2
        xc = jnp.logical_or(p == 1, p == 2).astype(my.dtype)

        def pos(px, py, pz):
            return 4 * pz + 2 * py + jnp.bitwise_xor(px, py)

        xn = pos(1 - xc, y, z)
        yn = pos(xc, 1 - y, z)
        zn = pos(xc, y, 1 - z)
        bd = pos(1 - xc, 1 - y, 1 - z)

        slot_origin = {
            SX: xn, SY: yn, SZ: zn,
            SBD: pos(1 - xc, 1 - y, 1 - z),
            SXY: pos(1 - xc, 1 - y, z),
            SYZ: pos(xc, 1 - y, 1 - z),
            SXZ: pos(1 - xc, y, 1 - z),
        }

        barrier_sem = pltpu.get_barrier_semaphore()
        for t in (xn, yn, zn, bd):
            pl.semaphore_signal(
                barrier_sem, inc=1,
                device_id=(t,), device_id_type=pl.DeviceIdType.MESH,
            )
        pl.semaphore_wait(barrier_sem, 4)

        own_ref[:, :] = x_ref[:, :].astype(jnp.bfloat16)

        A = pl.ds(0, H)
        B = pl.ds(H, H)

        def copy(src, dst_slot, half, sem_id, target):
            return pltpu.make_async_remote_copy(
                src_ref=src,
                dst_ref=recv_ref.at[dst_slot, half],
                send_sem=send_sems.at[sem_id],
                recv_sem=recv_sems.at[sem_id],
                device_id=(target,),
                device_id_type=pl.DeviceIdType.MESH,
            )

        FULL = pl.ds(0, M_PER)

        p1 = [
            copy(own_ref.at[A], SX, A, 0, xn),
            copy(own_ref.at[A], SY, A, 1, yn),
            copy(own_ref.at[A], SZ, A, 2, zn),
            copy(own_ref.at[B], SX, B, 3, xn),
            copy(own_ref.at[B], SY, B, 4, yn),
            copy(own_ref.at[B], SZ, B, 5, zn),
            copy(own_ref.at[FULL], SBD, FULL, 12, bd),
        ]
        for s in p1:
            s.start()

        w16 = w_ref[:, :].astype(jnp.bfloat16)

        def compute(src, origin_pos):
            yy = jnp.dot(src, w16, preferred_element_type=jnp.float32)
            out_ref[pl.ds(origin_pos * M_PER, M_PER), :] = _gelu(yy)

        compute(own_ref[:, :], my)

        sem_dst = {
            0: (SX, A), 1: (SY, A), 2: (SZ, A),
            3: (SX, B), 4: (SY, B), 5: (SZ, B),
            6: (SXY, A), 7: (SXY, B), 8: (SYZ, A), 9: (SYZ, B),
            10: (SXZ, A), 11: (SXZ, B), 12: (SBD, FULL),
        }

        def wait(sem_id):
            slot, half = sem_dst[sem_id]
            pltpu.make_async_remote_copy(
                src_ref=recv_ref.at[slot, half],
                dst_ref=recv_ref.at[slot, half],
                send_sem=send_sems.at[sem_id],
                recv_sem=recv_sems.at[sem_id],
                device_id=(my,),
                device_id_type=pl.DeviceIdType.MESH,
            ).wait_recv()

        p2 = []
        for sem_in, rdma_args in (
            (0, (recv_ref.at[SX, A], SXZ, A, 10, zn)),
            (1, (recv_ref.at[SY, A], SXY, A, 6, xn)),
            (2, (recv_ref.at[SZ, A], SYZ, A, 8, yn)),
            (3, (recv_ref.at[SX, B], SXY, B, 7, yn)),
            (4, (recv_ref.at[SY, B], SYZ, B, 9, zn)),
            (5, (recv_ref.at[SZ, B], SXZ, B, 11, xn)),
        ):
            wait(sem_in)
            r = copy(*rdma_args)
            r.start()
            p2.append(r)

        compute(recv_ref[SX], slot_origin[SX])
        compute(recv_ref[SY], slot_origin[SY])
        compute(recv_ref[SZ], slot_origin[SZ])

        wait(6)
        wait(7)
        compute(recv_ref[SXY], slot_origin[SXY])
        wait(8)
        wait(9)
        compute(recv_ref[SYZ], slot_origin[SYZ])
        wait(10)
        wait(11)
        compute(recv_ref[SXZ], slot_origin[SXZ])

        wait(12)
        compute(recv_ref[SBD], slot_origin[SBD])

        for s in p1 + p2:
            s.wait_send()

    return pl.pallas_call(
        body,
        out_shape=jax.ShapeDtypeStruct((N_DEV * M_PER, N_PER), jnp.float32),
        in_specs=[
            pl.BlockSpec(memory_space=pltpu.VMEM),
            pl.BlockSpec(memory_space=pltpu.VMEM),
        ],
        out_specs=pl.BlockSpec(memory_space=pltpu.VMEM),
        scratch_shapes=[
            pltpu.VMEM((M_PER, K), jnp.bfloat16),
            pltpu.VMEM((N_DEV - 1, M_PER, K), jnp.bfloat16),
            pltpu.SemaphoreType.DMA((13,)),
            pltpu.SemaphoreType.DMA((13,)),
        ],
        compiler_params=pltpu.CompilerParams(collective_id=0),
    )(x, w_mat)


# device time: 18131 ns/iter; 1.0409x vs baseline; 1.0409x over previous
import jax
import jax.numpy as jnp
from jax import lax
from jax.experimental import pallas as pl
from jax.experimental.pallas import tpu as pltpu

N_DEV = 8
M_PER = 128
H = 64
K = 1024
N_PER = 128

SX, SY, SZ, SBD, SXY, SYZ, SXZ = range(7)


def _gelu(y):
    c = 0.7978845608028654
    return 0.5 * y * (1.0 + jnp.tanh(c * (y + 0.044715 * y * y * y)))


def kernel(x, w_mat):
    def body(x_ref, w_ref, out_ref, own_ref, recv_ref, send_sems, recv_sems):
        my = lax.axis_index("i")

        z = my // 4
        p = my % 4
        y = p // 2
        xc = jnp.logical_or(p == 1, p == 2).astype(my.dtype)

        def pos(px, py, pz):
            return 4 * pz + 2 * py + jnp.bitwise_xor(px, py)

        xn = pos(1 - xc, y, z)
        yn = pos(xc, 1 - y, z)
        zn = pos(xc, y, 1 - z)
        xyd = pos(1 - xc, 1 - y, z)
        yzd = pos(xc, 1 - y, 1 - z)
        xzd = pos(1 - xc, y, 1 - z)
        bdd = pos(1 - xc, 1 - y, 1 - z)

        slot_origin = {
            SX: xn, SY: yn, SZ: zn, SBD: bdd,
            SXY: xyd, SYZ: yzd, SXZ: xzd,
        }

        barrier_sem = pltpu.get_barrier_semaphore()
        for t in (xn, yn, zn, yzd, xzd):
            pl.semaphore_signal(
                barrier_sem, inc=1,
                device_id=(t,), device_id_type=pl.DeviceIdType.MESH,
            )
        pl.semaphore_wait(barrier_sem, 5)

        own_ref[:, :] = x_ref[:, :].astype(jnp.bfloat16)

        A = pl.ds(0, H)
        B = pl.ds(H, H)

        def copy(src, dst_slot, half, sem_id, target):
            return pltpu.make_async_remote_copy(
                src_ref=src,
                dst_ref=recv_ref.at[dst_slot, half],
                send_sem=send_sems.at[sem_id],
                recv_sem=recv_sems.at[sem_id],
                device_id=(target,),
                device_id_type=pl.DeviceIdType.MESH,
            )

        p1 = [
            copy(own_ref.at[A], SX, A, 0, xn),
            copy(own_ref.at[A], SY, A, 1, yn),
            copy(own_ref.at[A], SZ, A, 2, zn),
            copy(own_ref.at[A], SYZ, A, 8, yzd),
            copy(own_ref.at[B], SXZ, B, 11, xzd),
            copy(own_ref.at[B], SX, B, 3, xn),
            copy(own_ref.at[B], SY, B, 4, yn),
            copy(own_ref.at[B], SZ, B, 5, zn),
        ]
        for s in p1:
            s.start()

        w16 = w_ref[:, :].astype(jnp.bfloat16)

        def compute(src, origin_pos):
            yy = jnp.dot(src, w16, preferred_element_type=jnp.float32)
            out_ref[pl.ds(origin_pos * M_PER, M_PER), :] = _gelu(yy)

        compute(own_ref[:, :], my)

        sem_dst = {
            0: (SX, A), 1: (SY, A), 2: (SZ, A),
            3: (SX, B), 4: (SY, B), 5: (SZ, B),
            6: (SXY, A), 7: (SXY, B), 8: (SYZ, A), 9: (SYZ, B),
            10: (SXZ, A), 11: (SXZ, B), 12: (SBD, A), 13: (SBD, B),
        }

        def wait(sem_id):
            slot, half = sem_dst[sem_id]
            pltpu.make_async_remote_copy(
                src_ref=recv_ref.at[slot, half],
                dst_ref=recv_ref.at[slot, half],
                send_sem=send_sems.at[sem_id],
                recv_sem=recv_sems.at[sem_id],
                device_id=(my,),
                device_id_type=pl.DeviceIdType.MESH,
            ).wait_recv()

        p2 = []
        for sem_in, rdma_args in (
            (0, (recv_ref.at[SX, A], SXZ, A, 10, zn)),
            (1, (recv_ref.at[SY, A], SXY, A, 6, xn)),
            (3, (recv_ref.at[SX, B], SXY, B, 7, yn)),
            (4, (recv_ref.at[SY, B], SYZ, B, 9, zn)),
        ):
            wait(sem_in)
            r = copy(*rdma_args)
            r.start()
            p2.append(r)

        wait(8)
        p3a = copy(recv_ref.at[SYZ, A], SBD, A, 12, xn)
        p3a.start()
        wait(11)
        p3b = copy(recv_ref.at[SXZ, B], SBD, B, 13, yn)
        p3b.start()

        compute(recv_ref[SX], slot_origin[SX])
        compute(recv_ref[SY], slot_origin[SY])
        wait(2)
        wait(5)
        compute(recv_ref[SZ], slot_origin[SZ])

        wait(6)
        wait(7)
        compute(recv_ref[SXY], slot_origin[SXY])
        wait(9)
        compute(recv_ref[SYZ], slot_origin[SYZ])
        wait(10)
        compute(recv_ref[SXZ], slot_origin[SXZ])

        wait(12)
        wait(13)
        compute(recv_ref[SBD], slot_origin[SBD])

        for s in p1 + p2 + [p3a, p3b]:
            s.wait_send()

    return pl.pallas_call(
        body,
        out_shape=jax.ShapeDtypeStruct((N_DEV * M_PER, N_PER), jnp.float32),
        in_specs=[
            pl.BlockSpec(memory_space=pltpu.VMEM),
            pl.BlockSpec(memory_space=pltpu.VMEM),
        ],
        out_specs=pl.BlockSpec(memory_space=pltpu.VMEM),
        scratch_shapes=[
            pltpu.VMEM((M_PER, K), jnp.bfloat16),
            pltpu.VMEM((N_DEV - 1, M_PER, K), jnp.bfloat16),
            pltpu.SemaphoreType.DMA((14,)),
            pltpu.SemaphoreType.DMA((14,)),
        ],
        compiler_params=pltpu.CompilerParams(collective_id=0),
    )(x, w_mat)


# device time: 16478 ns/iter; 1.1453x vs baseline; 1.1003x over previous
import jax
import jax.numpy as jnp
from jax import lax
from jax.experimental import pallas as pl
from jax.experimental.pallas import tpu as pltpu

N_DEV = 8
M_PER = 128
H = 64
K = 1024
N_PER = 128

SX, SY, SZ, SBD, SXY, SYZ, SXZ = range(7)


def _gelu(y):
    c = 0.7978845608028654
    return 0.5 * y * (1.0 + jnp.tanh(c * (y + 0.044715 * y * y * y)))


def kernel(x, w_mat):
    def body(x_ref, w_ref, out_ref, own_ref, recv_ref, send_sems, recv_sems):
        my = lax.axis_index("i")

        z = my // 4
        p = my % 4
        y = p // 2
        xc = jnp.logical_or(p == 1, p == 2).astype(my.dtype)

        def pos(px, py, pz):
            return 4 * pz + 2 * py + jnp.bitwise_xor(px, py)

        xn = pos(1 - xc, y, z)
        yn = pos(xc, 1 - y, z)
        zn = pos(xc, y, 1 - z)

        slot_origin = {
            SX: xn, SY: yn, SZ: zn,
            SBD: pos(1 - xc, 1 - y, 1 - z),
            SXY: pos(1 - xc, 1 - y, z),
            SYZ: pos(xc, 1 - y, 1 - z),
            SXZ: pos(1 - xc, y, 1 - z),
        }

        barrier_sem = pltpu.get_barrier_semaphore()
        for t in (xn, yn, zn):
            pl.semaphore_signal(
                barrier_sem, inc=1,
                device_id=(t,), device_id_type=pl.DeviceIdType.MESH,
            )
        own_ref[:, :] = x_ref[:, :].astype(jnp.bfloat16)
        w16 = w_ref[:, :].astype(jnp.bfloat16)
        pl.semaphore_wait(barrier_sem, 3)

        A = pl.ds(0, H)
        B = pl.ds(H, H)

        def copy(src, dst_slot, half, sem_id, target):
            return pltpu.make_async_remote_copy(
                src_ref=src,
                dst_ref=recv_ref.at[dst_slot, half],
                send_sem=send_sems.at[sem_id],
                recv_sem=recv_sems.at[sem_id],
                device_id=(target,),
                device_id_type=pl.DeviceIdType.MESH,
            )

        p1 = [
            copy(own_ref.at[A], SX, A, 0, xn),
            copy(own_ref.at[B], SY, B, 2, yn),
            copy(own_ref.at[A], SZ, A, 4, zn),
            copy(own_ref.at[B], SX, B, 1, xn),
            copy(own_ref.at[A], SY, A, 3, yn),
            copy(own_ref.at[B], SZ, B, 5, zn),
        ]
        for s in p1:
            s.start()

        def compute(src, origin_pos):
            yy = jnp.dot(src, w16, preferred_element_type=jnp.float32)
            out_ref[pl.ds(origin_pos * M_PER, M_PER), :] = _gelu(yy)

        compute(own_ref[:, :], my)

        sem_dst = {
            0: (SX, A), 1: (SX, B), 2: (SY, B), 3: (SY, A),
            4: (SZ, A), 5: (SZ, B),
            6: (SYZ, A), 7: (SXY, A), 8: (SYZ, B), 9: (SXZ, A),
            10: (SXY, B), 11: (SXZ, B), 12: (SBD, A), 13: (SBD, B),
        }

        def wait(sem_id):
            slot, half = sem_dst[sem_id]
            pltpu.make_async_remote_copy(
                src_ref=recv_ref.at[slot, half],
                dst_ref=recv_ref.at[slot, half],
                send_sem=send_sems.at[sem_id],
                recv_sem=recv_sems.at[sem_id],
                device_id=(my,),
                device_id_type=pl.DeviceIdType.MESH,
            ).wait_recv()

        fwds = []

        def fwd(src_slot, src_half, dst_slot, dst_half, sem_id, target):
            r = copy(recv_ref.at[src_slot, src_half], dst_slot, dst_half,
                     sem_id, target)
            r.start()
            fwds.append(r)

        wait(4)
        fwd(SZ, A, SYZ, A, 6, yn)
        wait(0)
        fwd(SX, A, SXY, A, 7, yn)
        fwd(SX, A, SXZ, A, 9, zn)
        wait(2)
        fwd(SY, B, SYZ, B, 8, zn)
        fwd(SY, B, SXY, B, 10, xn)
        wait(5)
        fwd(SZ, B, SXZ, B, 11, xn)

        wait(6)
        fwd(SYZ, A, SBD, A, 12, xn)
        wait(10)
        fwd(SXY, B, SBD, B, 13, zn)

        wait(1)
        compute(recv_ref[SX], slot_origin[SX])
        wait(3)
        compute(recv_ref[SY], slot_origin[SY])
        compute(recv_ref[SZ], slot_origin[SZ])

        wait(7)
        compute(recv_ref[SXY], slot_origin[SXY])
        wait(8)
        compute(recv_ref[SYZ], slot_origin[SYZ])
        wait(9)
        wait(11)
        compute(recv_ref[SXZ], slot_origin[SXZ])

        wait(12)
        wait(13)
        compute(recv_ref[SBD], slot_origin[SBD])

        for s in p1 + fwds:
            s.wait_send()

    return pl.pallas_call(
        body,
        out_shape=jax.ShapeDtypeStruct((N_DEV * M_PER, N_PER), jnp.float32),
        in_specs=[
            pl.BlockSpec(memory_space=pltpu.VMEM),
            pl.BlockSpec(memory_space=pltpu.VMEM),
        ],
        out_specs=pl.BlockSpec(memory_space=pltpu.VMEM),
        scratch_shapes=[
            pltpu.VMEM((M_PER, K), jnp.bfloat16),
            pltpu.VMEM((N_DEV - 1, M_PER, K), jnp.bfloat16),
            pltpu.SemaphoreType.DMA((14,)),
            pltpu.SemaphoreType.DMA((14,)),
        ],
        compiler_params=pltpu.CompilerParams(collective_id=0),
    )(x, w_mat)
